# baseline (speedup 1.0000x reference)
import functools

import jax
import jax.numpy as jnp
from jax import lax
from jax.experimental import pallas as pl
from jax.experimental.pallas import tpu as pltpu
from jax.experimental.pallas import tpu_sc as plsc

NC = 2
NS = 16
L = 16
NW = NC * NS
BLK = 128
PITCH = BLK + 1
CHUNK = 2
NBUF = 3


@functools.partial(jax.jit, static_argnames=("B", "F"))
def _gmf_sc(user_ids, item_ids, user_emb_t, item_emb_t, *, B, F):
    b_per_w = B // NW
    n_super = b_per_w // L
    mesh = plsc.VectorSubcoreMesh(core_axis_name="c", subcore_axis_name="s")

    @functools.partial(
        pl.kernel,
        out_type=jax.ShapeDtypeStruct((B,), jnp.float32),
        mesh=mesh,
        compiler_params=pltpu.CompilerParams(
            needs_layout_passes=False, use_tc_tiling_on_sc=True),
        scratch_types=[
            pltpu.VMEM((b_per_w,), jnp.int32),
            pltpu.VMEM((b_per_w,), jnp.int32),
            pltpu.VMEM((NBUF, CHUNK, F, PITCH), jnp.float32),
            pltpu.VMEM((NBUF, CHUNK, F, PITCH), jnp.float32),
            pltpu.VMEM((L * (L + 1),), jnp.float32),
            pltpu.VMEM((b_per_w,), jnp.float32),
            pltpu.SemaphoreType.DMA,
        ],
    )
    def gmf(uids_hbm, iids_hbm, uemb_t, iemb_t, out_hbm,
            idx_u, idx_i, bufs_u, bufs_i, scr, out_v, sem):
        wid = lax.axis_index("s") * NC + lax.axis_index("c")
        base = wid * b_per_w

        pltpu.sync_copy(uids_hbm.at[pl.ds(base, b_per_w)], idx_u)
        pltpu.sync_copy(iids_hbm.at[pl.ds(base, b_per_w)], idx_i)

        lane = lax.iota(jnp.int32, L)
        n_sub = L // CHUNK

        def superchunk(g, carry):
            iv_u = idx_u[pl.ds(g * L, L)]
            iv_i = idx_i[pl.ds(g * L, L)]

            def fire(sub):
                hs = []
                for t in range(CHUNK):
                    ru = iv_u[sub * CHUNK + t]
                    ri = iv_i[sub * CHUNK + t]
                    qu = pl.multiple_of((ru // BLK) * BLK, BLK)
                    qi = pl.multiple_of((ri // BLK) * BLK, BLK)
                    hs.append(pltpu.async_copy(
                        uemb_t.at[pl.ds(0, F), pl.ds(qu, BLK)],
                        bufs_u.at[sub % NBUF, t, pl.ds(0, F), pl.ds(0, BLK)],
                        sem))
                    hs.append(pltpu.async_copy(
                        iemb_t.at[pl.ds(0, F), pl.ds(qi, BLK)],
                        bufs_i.at[sub % NBUF, t, pl.ds(0, F), pl.ds(0, BLK)],
                        sem))
                return hs

            def consume(sub):
                for t in range(CHUNK):
                    ru = iv_u[sub * CHUNK + t]
                    ri = iv_i[sub * CHUNK + t]
                    su = ru % BLK
                    si = ri % BLK
                    bvec = jnp.full((L,), sub % NBUF, jnp.int32)
                    tvec = jnp.full((L,), t, jnp.int32)
                    suv = jnp.full((L,), su, jnp.int32)
                    siv = jnp.full((L,), si, jnp.int32)
                    u0 = plsc.load_gather(bufs_u, [bvec, tvec, lane, suv])
                    u1 = plsc.load_gather(bufs_u, [bvec, tvec, lane + L, suv])
                    i0 = plsc.load_gather(bufs_i, [bvec, tvec, lane, siv])
                    i1 = plsc.load_gather(bufs_i, [bvec, tvec, lane + L, siv])
                    prod = u0 * i0 + u1 * i1
                    row = sub * CHUNK + t
                    scr[pl.ds(row * (L + 1), L)] = prod

            inflight = [fire(s) for s in range(NBUF - 1)]
            for sub in range(n_sub):
                for h in inflight.pop(0):
                    h.wait()
                if sub + NBUF - 1 < n_sub:
                    inflight.append(fire(sub + NBUF - 1))
                consume(sub)

            acc = jnp.zeros((L,), jnp.float32)
            row_off = lane * (L + 1)
            for f in range(L):
                acc = acc + plsc.load_gather(scr, [row_off + f])
            out_v[pl.ds(g * L, L)] = jnp.maximum(acc, 0.0)
            return carry

        lax.fori_loop(0, n_super, superchunk, 0)

        pltpu.sync_copy(out_v, out_hbm.at[pl.ds(base, b_per_w)])

    return gmf(user_ids, item_ids, user_emb_t, item_emb_t)


def kernel(user_ids, item_ids, user_emb, item_emb):
    B = user_ids.shape[0]
    F = user_emb.shape[1]
    return _gmf_sc(user_ids.astype(jnp.int32), item_ids.astype(jnp.int32),
                   user_emb.T, item_emb.T, B=B, F=F)
